# Initial kernel scaffold; baseline (speedup 1.0000x reference)
import jax
import jax.numpy as jnp
from jax.experimental import pallas as pl


def kernel(x_1, edge_index_1, x_2, edge_index_2, W1_1, b1_1, W2_1, b2_1, W3_1, b3_1, Wend_1, bend_1, W1_2, b1_2, W2_2, b2_2, W3_2, b3_2, Wend_2, bend_2, lw1_W, lw1_b, lw2_W, lw2_b, lf_W, lf_b, out_W, out_b):
    raise NotImplementedError("write your pallas kernel here")


# speedup vs baseline: 8.5179x; 8.5179x over previous
import functools

import jax
import jax.numpy as jnp
from jax import lax
from jax.experimental import pallas as pl
from jax.experimental.pallas import tpu as pltpu
from jax.experimental.pallas import tpu_sc as plsc

N = 10000
E = 320000
H = 128

NC = 2
NS = 16
CH = 128

N_PAD = 10240
ROWS_PER_TILE = N_PAD // NS
ROW_COPIES = ROWS_PER_TILE // CH
CHUNKS = -(-E // CH)
CHUNKS_PER_TILE = -(-CHUNKS // NS)
E_PAD = NS * CHUNKS_PER_TILE * CH
DW = H

_mesh = plsc.VectorSubcoreMesh(core_axis_name="c", subcore_axis_name="s",
                               num_cores=NC, num_subcores=NS)


def _deg_body(dst_hbm, out_hbm, acc, didx, ones_v, zbuf):
    cid = lax.axis_index("c")
    sid = lax.axis_index("s")

    def _init(r, carry):
        for k in range(DW // 16):
            zbuf[r, pl.ds(k * 16, 16)] = jnp.zeros((16,), jnp.float32)
            ones_v[r, pl.ds(k * 16, 16)] = jnp.ones((16,), jnp.float32)
        return carry
    lax.fori_loop(0, CH, _init, 0)

    base = sid * ROWS_PER_TILE
    for j in range(ROW_COPIES):
        pltpu.sync_copy(zbuf, acc.at[pl.ds(base + j * CH, CH)])
    plsc.subcore_barrier()

    cbase = sid * CHUNKS_PER_TILE

    def _step(i, carry):
        off = (cbase + i) * CH
        pltpu.sync_copy(dst_hbm.at[cid, pl.ds(off, CH)], didx.at[0])
        pltpu.sync_copy(ones_v, acc.at[didx.at[0]], add=True)
        return carry
    lax.fori_loop(0, CHUNKS_PER_TILE, _step, 0)
    plsc.subcore_barrier()

    for j in range(ROW_COPIES):
        r0 = base + j * CH
        pltpu.sync_copy(acc.at[pl.ds(r0, CH)], zbuf)
        pltpu.sync_copy(zbuf, out_hbm.at[cid].at[pl.ds(r0, CH)])


_deg_sc = pl.kernel(
    _deg_body,
    out_type=jax.ShapeDtypeStruct((2, N_PAD, DW), jnp.float32),
    mesh=_mesh,
    scratch_types=[
        pltpu.VMEM_SHARED((N_PAD, DW), jnp.float32),
        pltpu.VMEM((1, CH), jnp.int32),
        pltpu.VMEM((CH, DW), jnp.float32),
        pltpu.VMEM((CH, DW), jnp.float32),
    ],
)


def _edge_sum_body(ys_hbm, src_hbm, dst_hbm, out_hbm, acc, sidx, didx, rows,
                   gsem):
    cid = lax.axis_index("c")
    sid = lax.axis_index("s")

    def _zero(r, carry):
        for k in range(H // 16):
            rows[0, r, pl.ds(k * 16, 16)] = jnp.zeros((16,), jnp.float32)
        return carry
    lax.fori_loop(0, CH, _zero, 0)

    base = sid * ROWS_PER_TILE
    for j in range(ROW_COPIES):
        pltpu.sync_copy(rows.at[0], acc.at[pl.ds(base + j * CH, CH)])
    plsc.subcore_barrier()

    cbase = sid * CHUNKS_PER_TILE

    def _step(i, carry):
        off = (cbase + i) * CH
        pltpu.sync_copy(src_hbm.at[cid, pl.ds(off, CH)], sidx.at[0])
        pltpu.sync_copy(dst_hbm.at[cid, pl.ds(off, CH)], didx.at[0])
        pltpu.async_copy(ys_hbm.at[cid].at[sidx.at[0]], rows.at[0],
                         gsem).wait()
        pltpu.sync_copy(rows.at[0], acc.at[didx.at[0]], add=True)
        return carry
    lax.fori_loop(0, CHUNKS_PER_TILE, _step, 0)
    plsc.subcore_barrier()

    for j in range(ROW_COPIES):
        r0 = base + j * CH
        pltpu.sync_copy(acc.at[pl.ds(r0, CH)], rows.at[0])
        pltpu.sync_copy(rows.at[0], out_hbm.at[cid].at[pl.ds(r0, CH)])


_edge_sum_sc = pl.kernel(
    _edge_sum_body,
    out_type=jax.ShapeDtypeStruct((2, N_PAD, H), jnp.float32),
    mesh=_mesh,
    scratch_types=[
        pltpu.VMEM_SHARED((N_PAD, H), jnp.float32),
        pltpu.VMEM((1, CH), jnp.int32),
        pltpu.VMEM((1, CH), jnp.int32),
        pltpu.VMEM((1, CH, H), jnp.float32),
        pltpu.SemaphoreType.DMA,
    ],
)


_PREC = lax.Precision.HIGHEST
BN = 1280
NBLK = N_PAD // BN


def _first_body(xs_ref, Ws_ref, degp_ref, ys_ref, dinv_ref):
    dinv = lax.rsqrt(1.0 + degp_ref[0, :, 0:1])
    dinv_ref[0] = dinv
    ys_ref[0] = jnp.dot(xs_ref[0], Ws_ref[0],
                        preferred_element_type=jnp.float32,
                        precision=_PREC) * dinv


def _tc_first(xs, Ws, degp):
    return pl.pallas_call(
        _first_body,
        grid=(2, NBLK),
        in_specs=[
            pl.BlockSpec((1, BN, H), lambda g, i: (g, i, 0)),
            pl.BlockSpec((1, H, H), lambda g, i: (g, 0, 0)),
            pl.BlockSpec((1, BN, DW), lambda g, i: (g, i, 0)),
        ],
        out_specs=[
            pl.BlockSpec((1, BN, H), lambda g, i: (g, i, 0)),
            pl.BlockSpec((1, BN, 1), lambda g, i: (g, i, 0)),
        ],
        out_shape=[
            jax.ShapeDtypeStruct((2, N_PAD, H), jnp.float32),
            jax.ShapeDtypeStruct((2, N_PAD, 1), jnp.float32),
        ],
    )(xs, Ws, degp)


def _mid_body(ys_ref, acc_ref, dinv_ref, b_ref, W_ref, out_ref):
    dinv = dinv_ref[0]
    h = jnp.maximum(dinv * (ys_ref[0] + acc_ref[0]) + b_ref[0], 0.0)
    out_ref[0] = jnp.dot(h, W_ref[0],
                         preferred_element_type=jnp.float32,
                         precision=_PREC) * dinv


def _tc_mid(ys, acc, dinv2, bs, Ws):
    return pl.pallas_call(
        _mid_body,
        grid=(2, NBLK),
        in_specs=[
            pl.BlockSpec((1, BN, H), lambda g, i: (g, i, 0)),
            pl.BlockSpec((1, BN, H), lambda g, i: (g, i, 0)),
            pl.BlockSpec((1, BN, 1), lambda g, i: (g, i, 0)),
            pl.BlockSpec((1, 1, H), lambda g, i: (g, 0, 0)),
            pl.BlockSpec((1, H, H), lambda g, i: (g, 0, 0)),
        ],
        out_specs=pl.BlockSpec((1, BN, H), lambda g, i: (g, i, 0)),
        out_shape=jax.ShapeDtypeStruct((2, N_PAD, H), jnp.float32),
    )(ys, acc, dinv2, bs, Ws)


def _final_body(ys_ref, acc_ref, dinv_ref, bend_ref, lw1W_ref, lw1b_ref,
                lw2W_ref, lw2b_ref, lfW_ref, lfb_ref, outW_ref, outb_ref,
                o_ref):
    x1 = dinv_ref[0] * (ys_ref[0] + acc_ref[0]) + bend_ref[0]
    x2 = dinv_ref[1] * (ys_ref[1] + acc_ref[1]) + bend_ref[1]
    s1 = jnp.sum(x1 * lw1W_ref[:, 0][None, :], axis=1, keepdims=True)
    s2 = jnp.sum(x2 * lw2W_ref[:, 0][None, :], axis=1, keepdims=True)
    f1 = jax.nn.sigmoid(s1 + lw1b_ref[0, 0])
    f2 = jax.nn.sigmoid(s2 + lw2b_ref[0, 0])
    f1n = f1 / (f1 + f2)
    v = f1n * x1 + (1.0 - f1n) * x2
    o = jnp.maximum(
        jnp.dot(v, lfW_ref[...], preferred_element_type=jnp.float32,
                precision=_PREC) + lfb_ref[0], 0.0)
    s3 = jnp.sum(o * outW_ref[:, 0][None, :], axis=1, keepdims=True)
    o_ref[...] = jax.nn.sigmoid(s3 + outb_ref[0, 0])


def _tc_final(ys, acc, dinv2, bends, lw1_W, lw1_b, lw2_W, lw2_b, lf_W, lf_b,
              out_W, out_b):
    full = lambda shape: pl.BlockSpec(shape, lambda i: tuple(0 for _ in shape))
    return pl.pallas_call(
        _final_body,
        grid=(NBLK,),
        in_specs=[
            pl.BlockSpec((2, BN, H), lambda i: (0, i, 0)),
            pl.BlockSpec((2, BN, H), lambda i: (0, i, 0)),
            pl.BlockSpec((2, BN, 1), lambda i: (0, i, 0)),
            full((2, 1, H)),
            full((H, 1)), full((1, 1)),
            full((H, 1)), full((1, 1)),
            full((H, H)), full((1, H)),
            full((H, 1)), full((1, 1)),
        ],
        out_specs=pl.BlockSpec((BN, 1), lambda i: (i, 0)),
        out_shape=jax.ShapeDtypeStruct((N_PAD, 1), jnp.float32),
    )(ys, acc, dinv2, bends, lw1_W, lw1_b, lw2_W, lw2_b, lf_W, lf_b,
      out_W, out_b)


def _pad_edges(ei):
    pad = jnp.full((E_PAD - E,), N, dtype=jnp.int32)
    src = jnp.concatenate([ei[0].astype(jnp.int32), pad])
    dst = jnp.concatenate([ei[1].astype(jnp.int32), pad])
    return src, dst


def kernel(x_1, edge_index_1, x_2, edge_index_2, W1_1, b1_1, W2_1, b2_1,
           W3_1, b3_1, Wend_1, bend_1, W1_2, b1_2, W2_2, b2_2, W3_2, b3_2,
           Wend_2, bend_2, lw1_W, lw1_b, lw2_W, lw2_b, lf_W, lf_b, out_W,
           out_b):
    xs = jnp.stack([
        jnp.pad(x_1, ((0, N_PAD - N), (0, 0))),
        jnp.pad(x_2, ((0, N_PAD - N), (0, 0))),
    ])
    s1, d1 = _pad_edges(edge_index_1)
    s2, d2 = _pad_edges(edge_index_2)
    srcs = jnp.stack([s1, s2])
    dsts = jnp.stack([d1, d2])

    W1s = jnp.stack([W1_1, W1_2])
    Wmids = [jnp.stack([W2_1, W2_2]), jnp.stack([W3_1, W3_2]),
             jnp.stack([Wend_1, Wend_2])]
    bmids = [jnp.stack([b1_1, b1_2]).reshape(2, 1, H),
             jnp.stack([b2_1, b2_2]).reshape(2, 1, H),
             jnp.stack([b3_1, b3_2]).reshape(2, 1, H)]
    bends = jnp.stack([bend_1, bend_2]).reshape(2, 1, H)

    degp = _deg_sc(dsts)
    ys, dinv2 = _tc_first(xs, W1s, degp)
    for W_l, b_l in zip(Wmids, bmids):
        acc = _edge_sum_sc(ys, srcs, dsts)
        ys = _tc_mid(ys, acc, dinv2, b_l, W_l)
    acc = _edge_sum_sc(ys, srcs, dsts)
    out = _tc_final(ys, acc, dinv2, bends, lw1_W, lw1_b.reshape(1, 1),
                    lw2_W, lw2_b.reshape(1, 1), lf_W, lf_b.reshape(1, H),
                    out_W, out_b.reshape(1, 1))
    return out[:N, 0]
